# baseline (device time: 67679 ns/iter reference)
import jax
import jax.numpy as jnp
from jax import lax
from jax.experimental import pallas as pl
from jax.experimental.pallas import tpu as pltpu

B, S, HL, D = 2, 1024, 16, 64
K = HL * D
N = 2048
S_HALF = S // 2


def kernel(O, Wo):
    O = O.reshape(B, S, K)

    def body(o_ref, w_ref, out_ref, send_buf, recv_buf, send_sem, recv_sem):
        my_x = lax.axis_index("x")
        my_y = lax.axis_index("y")
        my_z = lax.axis_index("z")
        partner = 1 - my_x

        barrier_sem = pltpu.get_barrier_semaphore()
        pl.semaphore_signal(
            barrier_sem, inc=1,
            device_id=(partner, my_y, my_z),
            device_id_type=pl.DeviceIdType.MESH,
        )
        pl.semaphore_wait(barrier_sem, 1)

        w16 = w_ref[...].astype(jnp.bfloat16)

        for b in range(B):
            lhs = o_ref[b, pl.ds(partner * S_HALF, S_HALF), :].astype(jnp.bfloat16)
            acc = lax.dot_general(
                lhs, w16, (((1,), (0,)), ((), ())),
                preferred_element_type=jnp.float32,
            )
            send_buf[b] = acc.astype(jnp.bfloat16)

        rdma = pltpu.make_async_remote_copy(
            src_ref=send_buf,
            dst_ref=recv_buf,
            send_sem=send_sem,
            recv_sem=recv_sem,
            device_id=(partner, my_y, my_z),
            device_id_type=pl.DeviceIdType.MESH,
        )
        rdma.start()

        for b in range(B):
            lhs = o_ref[b, pl.ds(my_x * S_HALF, S_HALF), :].astype(jnp.bfloat16)
            acc = lax.dot_general(
                lhs, w16, (((1,), (0,)), ((), ())),
                preferred_element_type=jnp.float32,
            )
            out_ref[b] = acc

        rdma.wait()
        out_ref[...] += recv_buf[...].astype(jnp.float32)

    return pl.pallas_call(
        body,
        out_shape=jax.ShapeDtypeStruct((B, S_HALF, N), jnp.float32),
        in_specs=[
            pl.BlockSpec(memory_space=pltpu.VMEM),
            pl.BlockSpec(memory_space=pltpu.VMEM),
        ],
        out_specs=pl.BlockSpec(memory_space=pltpu.VMEM),
        scratch_shapes=[
            pltpu.VMEM((B, S_HALF, N), jnp.bfloat16),
            pltpu.VMEM((B, S_HALF, N), jnp.bfloat16),
            pltpu.SemaphoreType.DMA,
            pltpu.SemaphoreType.DMA,
        ],
        compiler_params=pltpu.CompilerParams(collective_id=0),
    )(O, Wo)


# device time: 63169 ns/iter; 1.0714x vs baseline; 1.0714x over previous
import jax
import jax.numpy as jnp
from jax import lax
from jax.experimental import pallas as pl
from jax.experimental.pallas import tpu as pltpu

B, S, HL, D = 2, 1024, 16, 64
K = HL * D
N = 2048
S_HALF = S // 2
CH = 4
SC = S_HALF // CH
NCHUNK = B * CH


def kernel(O, Wo):
    O = O.reshape(B, S, K)

    def body(o_ref, w_ref, out_ref, send_buf, recv_buf, send_sems, recv_sems):
        my_x = lax.axis_index("x")
        my_y = lax.axis_index("y")
        my_z = lax.axis_index("z")
        partner = 1 - my_x

        barrier_sem = pltpu.get_barrier_semaphore()
        pl.semaphore_signal(
            barrier_sem, inc=1,
            device_id=(partner, my_y, my_z),
            device_id_type=pl.DeviceIdType.MESH,
        )
        pl.semaphore_wait(barrier_sem, 1)

        w16 = w_ref[...].astype(jnp.bfloat16)

        rdmas = []
        for b in range(B):
            for c in range(CH):
                i = b * CH + c
                lhs = o_ref[
                    b, pl.ds(partner * S_HALF + c * SC, SC), :
                ].astype(jnp.bfloat16)
                acc = lax.dot_general(
                    lhs, w16, (((1,), (0,)), ((), ())),
                    preferred_element_type=jnp.float32,
                )
                send_buf[b, c * SC:(c + 1) * SC] = acc.astype(jnp.bfloat16)
                rdma = pltpu.make_async_remote_copy(
                    src_ref=send_buf.at[b, pl.ds(c * SC, SC)],
                    dst_ref=recv_buf.at[b, pl.ds(c * SC, SC)],
                    send_sem=send_sems.at[i],
                    recv_sem=recv_sems.at[i],
                    device_id=(partner, my_y, my_z),
                    device_id_type=pl.DeviceIdType.MESH,
                )
                rdma.start()
                rdmas.append(rdma)

        for b in range(B):
            lhs = o_ref[b, pl.ds(my_x * S_HALF, S_HALF), :].astype(jnp.bfloat16)
            acc = lax.dot_general(
                lhs, w16, (((1,), (0,)), ((), ())),
                preferred_element_type=jnp.float32,
            )
            out_ref[b] = acc

        for b in range(B):
            for c in range(CH):
                i = b * CH + c
                rdmas[i].wait_recv()
                out_ref[b, c * SC:(c + 1) * SC] += recv_buf[
                    b, c * SC:(c + 1) * SC
                ].astype(jnp.float32)

        for rdma in rdmas:
            rdma.wait_send()

    return pl.pallas_call(
        body,
        out_shape=jax.ShapeDtypeStruct((B, S_HALF, N), jnp.float32),
        in_specs=[
            pl.BlockSpec(memory_space=pltpu.VMEM),
            pl.BlockSpec(memory_space=pltpu.VMEM),
        ],
        out_specs=pl.BlockSpec(memory_space=pltpu.VMEM),
        scratch_shapes=[
            pltpu.VMEM((B, S_HALF, N), jnp.bfloat16),
            pltpu.VMEM((B, S_HALF, N), jnp.bfloat16),
            pltpu.SemaphoreType.DMA((NCHUNK,)),
            pltpu.SemaphoreType.DMA((NCHUNK,)),
        ],
        compiler_params=pltpu.CompilerParams(collective_id=0),
    )(O, Wo)


# device time: 63168 ns/iter; 1.0714x vs baseline; 1.0000x over previous
import jax
import jax.numpy as jnp
from jax import lax
from jax.experimental import pallas as pl
from jax.experimental.pallas import tpu as pltpu

B, S, HL, D = 2, 1024, 16, 64
K = HL * D
N = 2048
S_HALF = S // 2
CH = 4
SC = S_HALF // CH
NCHUNK = B * CH


def kernel(O, Wo):
    OT = O.transpose(0, 2, 3, 1).reshape(B, K, S)

    def body(ot_ref, w_ref, out_hbm, out_vmem, send_buf, recv_buf,
             send_sems, recv_sems, store_sems):
        my_x = lax.axis_index("x")
        my_y = lax.axis_index("y")
        my_z = lax.axis_index("z")
        partner = 1 - my_x

        barrier_sem = pltpu.get_barrier_semaphore()
        pl.semaphore_signal(
            barrier_sem, inc=1,
            device_id=(partner, my_y, my_z),
            device_id_type=pl.DeviceIdType.MESH,
        )
        pl.semaphore_wait(barrier_sem, 1)

        w16 = w_ref[...].astype(jnp.bfloat16)

        rdmas = []
        for b in range(B):
            for c in range(CH):
                i = b * CH + c
                lhs = ot_ref[
                    b, :, pl.ds(partner * S_HALF + c * SC, SC)
                ].astype(jnp.bfloat16)
                acc = lax.dot_general(
                    lhs, w16, (((0,), (0,)), ((), ())),
                    preferred_element_type=jnp.float32,
                )
                send_buf[b, c * SC:(c + 1) * SC] = acc.astype(jnp.bfloat16)
                rdma = pltpu.make_async_remote_copy(
                    src_ref=send_buf.at[b, pl.ds(c * SC, SC)],
                    dst_ref=recv_buf.at[b, pl.ds(c * SC, SC)],
                    send_sem=send_sems.at[i],
                    recv_sem=recv_sems.at[i],
                    device_id=(partner, my_y, my_z),
                    device_id_type=pl.DeviceIdType.MESH,
                )
                rdma.start()
                rdmas.append(rdma)

        for b in range(B):
            lhs = ot_ref[b, :, pl.ds(my_x * S_HALF, S_HALF)].astype(jnp.bfloat16)
            acc = lax.dot_general(
                lhs, w16, (((0,), (0,)), ((), ())),
                preferred_element_type=jnp.float32,
            )
            out_vmem[b] = acc

        stores = []
        for b in range(B):
            for c in range(CH):
                i = b * CH + c
                rdmas[i].wait_recv()
                sl = slice(c * SC, (c + 1) * SC)
                out_vmem[b, sl] += recv_buf[b, sl].astype(jnp.float32)
                store = pltpu.make_async_copy(
                    out_vmem.at[b, sl], out_hbm.at[b, sl], store_sems.at[i]
                )
                store.start()
                stores.append(store)

        for rdma in rdmas:
            rdma.wait_send()
        for store in stores:
            store.wait()

    return pl.pallas_call(
        body,
        out_shape=jax.ShapeDtypeStruct((B, S_HALF, N), jnp.float32),
        in_specs=[
            pl.BlockSpec(memory_space=pltpu.VMEM),
            pl.BlockSpec(memory_space=pltpu.VMEM),
        ],
        out_specs=pl.BlockSpec(memory_space=pl.ANY),
        scratch_shapes=[
            pltpu.VMEM((B, S_HALF, N), jnp.float32),
            pltpu.VMEM((B, S_HALF, N), jnp.bfloat16),
            pltpu.VMEM((B, S_HALF, N), jnp.bfloat16),
            pltpu.SemaphoreType.DMA((NCHUNK,)),
            pltpu.SemaphoreType.DMA((NCHUNK,)),
            pltpu.SemaphoreType.DMA((NCHUNK,)),
        ],
        compiler_params=pltpu.CompilerParams(
            collective_id=0,
            vmem_limit_bytes=100 * 1024 * 1024,
        ),
    )(OT, Wo)
